# baseline (speedup 1.0000x reference)
import jax
import jax.numpy as jnp
from jax import lax
from jax.experimental import pallas as pl
from jax.experimental.pallas import tpu as pltpu
from jax.experimental.pallas import tpu_sc as plsc

_KEEP = 0.9
_INV_KEEP = 1.0 / _KEEP
_NNZ = 4294967

_M32 = 0xFFFFFFFF
_ROTS = ((13, 15, 26, 6), (17, 29, 16, 24))


def _fold_key():
    def rotl(x, d):
        return ((x << d) | (x >> (32 - d))) & _M32

    ks = (0, 0, 0x1BD11BDA)
    inj = ((1, 2), (2, 0), (0, 1), (1, 2), (2, 0))
    x0, x1 = 0, 1
    for g in range(5):
        for r in _ROTS[g % 2]:
            x0 = (x0 + x1) & _M32
            x1 = rotl(x1, r) ^ x0
        a, b = inj[g]
        x0 = (x0 + ks[a]) & _M32
        x1 = (x1 + ks[b] + g + 1) & _M32
    return x0, x1


_K0, _K1 = _fold_key()
_KS2 = _K0 ^ _K1 ^ 0x1BD11BDA
_INJ = (
    (_K1, (_KS2 + 1) & _M32),
    (_KS2, (_K0 + 2) & _M32),
    (_K0, (_K1 + 3) & _M32),
    (_K1, (_KS2 + 4) & _M32),
    (_KS2, (_K0 + 5) & _M32),
)
_BITS_THRESH = 0x19999A00


def _keep_mask(idx_u32):
    x0 = jnp.full(idx_u32.shape, _K0, jnp.uint32)
    x1 = idx_u32 + jnp.uint32(_K1)
    for g in range(5):
        for r in _ROTS[g % 2]:
            x0 = x0 + x1
            x1 = (
                lax.shift_left(x1, jnp.uint32(r))
                | lax.shift_right_logical(x1, jnp.uint32(32 - r))
            ) ^ x0
        a, b = _INJ[g]
        x0 = x0 + jnp.uint32(a)
        x1 = x1 + jnp.uint32(b)
    return (x0 ^ x1) >= jnp.uint32(_BITS_THRESH)


_T = 16384
_NTF = _NNZ // _T
_TS = 208
_S = _TS * _T
_TAIL_OFF = _NTF * _T
_TAIL = _NNZ - _TAIL_OFF
_TAIL_NV = -(-_TAIL // 16)
_NW = 32


_CI = 134272
_CIL = 132480
_IND_CUT = 31 * _CI + _CIL


def _sc_body(vals, ind, out, ind_out, vbuf, isem):
    wid = lax.axis_index("s") * 2 + lax.axis_index("c")

    c0 = wid * _CI

    @pl.when(wid < _NW - 1)
    def _ind_start():
        pltpu.make_async_copy(
            ind.at[:, pl.ds(c0, _CI)], ind_out.at[:, pl.ds(c0, _CI)], isem
        ).start()

    @pl.when(wid == _NW - 1)
    def _ind_start_last():
        pltpu.make_async_copy(
            ind.at[:, pl.ds(c0, _CIL)], ind_out.at[:, pl.ds(c0, _CIL)], isem
        ).start()

    def do_tile(base, n_valid, nv):
        pltpu.sync_copy(vals.at[pl.ds(base, n_valid)], vbuf.at[pl.ds(0, n_valid)])
        lane = lax.broadcasted_iota(jnp.int32, (16,), 0)

        def one(off):
            idx = (base + off + lane).astype(jnp.uint32)
            keep = _keep_mask(idx)
            v16 = vbuf[pl.ds(off, 16)]
            vbuf[pl.ds(off, 16)] = jnp.where(
                keep, v16 * jnp.float32(_INV_KEEP), jnp.float32(0.0)
            )

        def vstep(v, carry):
            off = v * 32
            one(off)
            one(off + 16)
            return carry

        lax.fori_loop(0, nv // 2, vstep, None)
        pltpu.sync_copy(
            vbuf.at[pl.ds(0, n_valid)], out.at[pl.ds(base - _S, n_valid)]
        )

    nt = (jnp.int32(_NTF - _TS + _NW - 1) - wid) // _NW

    def tile_step(k, carry):
        do_tile((_TS + wid + k * _NW) * _T, _T, _T // 16)
        return carry

    lax.fori_loop(0, nt, tile_step, None)

    @pl.when(wid == (_NTF - _TS) % _NW)
    def _tail():
        do_tile(jnp.int32(_TAIL_OFF), _TAIL, _TAIL_NV)

    @pl.when(wid < _NW - 1)
    def _ind_wait():
        pltpu.make_async_copy(
            ind.at[:, pl.ds(c0, _CI)], ind_out.at[:, pl.ds(c0, _CI)], isem
        ).wait()

    @pl.when(wid == _NW - 1)
    def _ind_wait_last():
        pltpu.make_async_copy(
            ind.at[:, pl.ds(c0, _CIL)], ind_out.at[:, pl.ds(c0, _CIL)], isem
        ).wait()


_sc_tail = pl.kernel(
    _sc_body,
    out_type=(
        jax.ShapeDtypeStruct((_NNZ - _S,), jnp.float32),
        jax.ShapeDtypeStruct((2, _NNZ), jnp.int32),
    ),
    mesh=plsc.VectorSubcoreMesh(core_axis_name="c", subcore_axis_name="s"),
    scratch_types=[pltpu.VMEM((_T,), jnp.float32), pltpu.SemaphoreType.DMA],
)

_BLK = 262144
_SUB = _BLK // 128
_NB = _S // _BLK


def _tc_body(vals_ref, out_ref):
    b = pl.program_id(0)
    x = vals_ref[...].reshape(_SUB, 128)
    idx = (
        b * _BLK
        + lax.broadcasted_iota(jnp.int32, (_SUB, 128), 0) * 128
        + lax.broadcasted_iota(jnp.int32, (_SUB, 128), 1)
    ).astype(jnp.uint32)
    keep = _keep_mask(idx)
    y = jnp.where(keep, x * jnp.float32(_INV_KEEP), jnp.float32(0.0))
    out_ref[...] = y.reshape(_BLK)


def _tc_head(vals):
    return pl.pallas_call(
        _tc_body,
        grid=(_NB,),
        in_specs=[pl.BlockSpec((_BLK,), lambda b: (b,))],
        out_specs=pl.BlockSpec((_BLK,), lambda b: (b,)),
        out_shape=jax.ShapeDtypeStruct((_NNZ,), jnp.float32),
    )(vals)


def kernel(x_indices, x_values):
    tail, ind_sc = _sc_tail(x_values, x_indices)
    head = _tc_head(x_values)
    out = lax.dynamic_update_slice(head, tail, (_S,))
    ind_out = lax.dynamic_update_slice(
        ind_sc, lax.slice(x_indices, (0, _IND_CUT), (2, _NNZ)), (0, _IND_CUT)
    )
    return ind_out, out


# speedup vs baseline: 13.5509x; 13.5509x over previous
import jax
import jax.numpy as jnp
from jax import lax
from jax.experimental import pallas as pl
from jax.experimental.pallas import tpu as pltpu
from jax.experimental.pallas import tpu_sc as plsc

_KEEP = 0.9
_INV_KEEP = 1.0 / _KEEP
_NNZ = 4294967

_M32 = 0xFFFFFFFF
_ROTS = ((13, 15, 26, 6), (17, 29, 16, 24))


def _fold_key():
    def rotl(x, d):
        return ((x << d) | (x >> (32 - d))) & _M32

    ks = (0, 0, 0x1BD11BDA)
    inj = ((1, 2), (2, 0), (0, 1), (1, 2), (2, 0))
    x0, x1 = 0, 1
    for g in range(5):
        for r in _ROTS[g % 2]:
            x0 = (x0 + x1) & _M32
            x1 = rotl(x1, r) ^ x0
        a, b = inj[g]
        x0 = (x0 + ks[a]) & _M32
        x1 = (x1 + ks[b] + g + 1) & _M32
    return x0, x1


_K0, _K1 = _fold_key()
_KS2 = _K0 ^ _K1 ^ 0x1BD11BDA
_INJ = (
    (_K1, (_KS2 + 1) & _M32),
    (_KS2, (_K0 + 2) & _M32),
    (_K0, (_K1 + 3) & _M32),
    (_K1, (_KS2 + 4) & _M32),
    (_KS2, (_K0 + 5) & _M32),
)
_BITS_THRESH = 0x19999A00


def _keep_mask(idx_u32):
    x0 = jnp.full(idx_u32.shape, _K0, jnp.uint32)
    x1 = idx_u32 + jnp.uint32(_K1)
    for g in range(5):
        for r in _ROTS[g % 2]:
            x0 = x0 + x1
            x1 = (
                lax.shift_left(x1, jnp.uint32(r))
                | lax.shift_right_logical(x1, jnp.uint32(32 - r))
            ) ^ x0
        a, b = _INJ[g]
        x0 = x0 + jnp.uint32(a)
        x1 = x1 + jnp.uint32(b)
    return (x0 ^ x1) >= jnp.uint32(_BITS_THRESH)


_T = 16384
_NTF = _NNZ // _T
_TS = 208
_S = _TS * _T
_TAIL_OFF = _NTF * _T
_TAIL = _NNZ - _TAIL_OFF
_TAIL_NV = -(-_TAIL // 16)
_NW = 32


def _sc_body(vals, out, vbuf):
    wid = lax.axis_index("s") * 2 + lax.axis_index("c")

    def do_tile(base, n_valid, nv):
        pltpu.sync_copy(vals.at[pl.ds(base, n_valid)], vbuf.at[pl.ds(0, n_valid)])
        lane = lax.broadcasted_iota(jnp.int32, (16,), 0)

        def one(off):
            idx = (base + off + lane).astype(jnp.uint32)
            keep = _keep_mask(idx)
            v16 = vbuf[pl.ds(off, 16)]
            vbuf[pl.ds(off, 16)] = jnp.where(
                keep, v16 * jnp.float32(_INV_KEEP), jnp.float32(0.0)
            )

        def vstep(v, carry):
            off = v * 32
            one(off)
            one(off + 16)
            return carry

        lax.fori_loop(0, nv // 2, vstep, None)
        pltpu.sync_copy(
            vbuf.at[pl.ds(0, n_valid)], out.at[pl.ds(base - _S, n_valid)]
        )

    nt = (jnp.int32(_NTF - _TS + _NW - 1) - wid) // _NW

    def tile_step(k, carry):
        do_tile((_TS + wid + k * _NW) * _T, _T, _T // 16)
        return carry

    lax.fori_loop(0, nt, tile_step, None)

    @pl.when(wid == (_NTF - _TS) % _NW)
    def _tail():
        do_tile(jnp.int32(_TAIL_OFF), _TAIL, _TAIL_NV)


_sc_tail = pl.kernel(
    _sc_body,
    out_type=jax.ShapeDtypeStruct((_NNZ - _S,), jnp.float32),
    mesh=plsc.VectorSubcoreMesh(core_axis_name="c", subcore_axis_name="s"),
    scratch_types=[pltpu.VMEM((_T,), jnp.float32)],
)

_BLK = 262144
_SUB = _BLK // 128
_NB = _S // _BLK
_IBLK = 330496


def _tc_body(vals_ref, ind_ref, out_ref, ind_out_ref):
    b = pl.program_id(0)
    x = vals_ref[...].reshape(_SUB, 128)
    idx = (
        b * _BLK
        + lax.broadcasted_iota(jnp.int32, (_SUB, 128), 0) * 128
        + lax.broadcasted_iota(jnp.int32, (_SUB, 128), 1)
    ).astype(jnp.uint32)
    keep = _keep_mask(idx)
    y = jnp.where(keep, x * jnp.float32(_INV_KEEP), jnp.float32(0.0))
    out_ref[...] = y.reshape(_BLK)
    ind_out_ref[...] = ind_ref[...]


def _tc_head(vals, indices):
    return pl.pallas_call(
        _tc_body,
        grid=(_NB,),
        in_specs=[
            pl.BlockSpec((_BLK,), lambda b: (b,)),
            pl.BlockSpec((2, _IBLK), lambda b: (0, b)),
        ],
        out_specs=[
            pl.BlockSpec((_BLK,), lambda b: (b,)),
            pl.BlockSpec((2, _IBLK), lambda b: (0, b)),
        ],
        out_shape=[
            jax.ShapeDtypeStruct((_NNZ,), jnp.float32),
            jax.ShapeDtypeStruct((2, _NNZ), jnp.int32),
        ],
    )(vals, indices)


def kernel(x_indices, x_values):
    tail = _sc_tail(x_values)
    head, ind_out = _tc_head(x_values, x_indices)
    out = lax.dynamic_update_slice(head, tail, (_S,))
    return ind_out, out
